# Initial kernel scaffold; baseline (speedup 1.0000x reference)
import functools

import jax
import jax.numpy as jnp
from jax import lax
from jax.experimental import pallas as pl
from jax.experimental.pallas import tpu as pltpu
from jax.experimental.pallas import tpu_sc as plsc

N = 10000
E = 320000
D = 128

NC = 2
NS = 16
NW = NC * NS
CH = 128
NCH = -(-E // (NW * CH))
E_PAD = NW * NCH * CH
N_PAD = 10240
RPT = N_PAD // NS
RC = RPT // 128
GRID = N_PAD // 128

_mesh = plsc.VectorSubcoreMesh(core_axis_name="c", subcore_axis_name="s")


@functools.partial(
    pl.kernel,
    out_type=jax.ShapeDtypeStruct((NC, N_PAD, D), jnp.float32),
    mesh=_mesh,
    scratch_types=[
        pltpu.VMEM((NCH, CH), jnp.int32),
        pltpu.VMEM((NCH, CH), jnp.int32),
        pltpu.VMEM((CH, D), jnp.float32),
        pltpu.VMEM((CH, D), jnp.float32),
        pltpu.VMEM_SHARED((N_PAD, D), jnp.float32),
        pltpu.SemaphoreType.DMA,
    ],
)
def _sc_scatter(hs_hbm, src_hbm, dst_hbm, zer_hbm, out_hbm,
                src_v, dst_v, rows_v, zb_v, acc, gsem):
    c = lax.axis_index("c")
    s = lax.axis_index("s")
    wid = c * NS + s

    pltpu.sync_copy(src_hbm.at[wid], src_v)
    pltpu.sync_copy(dst_hbm.at[wid], dst_v)
    pltpu.sync_copy(zer_hbm, zb_v)

    @pl.loop(0, RC)
    def _(k):
        pltpu.sync_copy(zb_v, acc.at[pl.ds(s * RPT + k * 128, 128)])

    plsc.subcore_barrier()

    @pl.loop(0, NCH)
    def _(k):
        pltpu.async_copy(hs_hbm.at[src_v.at[k]], rows_v, gsem).wait()
        pltpu.sync_copy(rows_v, acc.at[dst_v.at[k]], add=True)

    plsc.subcore_barrier()

    @pl.loop(0, RC)
    def _(k):
        r = s * RPT + k * 128
        pltpu.sync_copy(acc.at[pl.ds(r, 128)], out_hbm.at[c, pl.ds(r, 128)])


@functools.partial(
    pl.kernel,
    out_type=jax.ShapeDtypeStruct((NC, N_PAD, 1), jnp.float32),
    mesh=_mesh,
    scratch_types=[
        pltpu.VMEM((NCH, CH), jnp.int32),
        pltpu.VMEM((CH, 1), jnp.float32),
        pltpu.VMEM((RPT, 1), jnp.float32),
        pltpu.VMEM_SHARED((N_PAD, 1), jnp.float32),
        pltpu.SemaphoreType.DMA,
    ],
)
def _sc_degree(dst_hbm, one_hbm, zer_hbm, out_hbm, dst_v, one_v, zb_v, acc, _sem):
    c = lax.axis_index("c")
    s = lax.axis_index("s")
    wid = c * NS + s

    pltpu.sync_copy(dst_hbm.at[wid], dst_v)
    pltpu.sync_copy(one_hbm, one_v)
    pltpu.sync_copy(zer_hbm, zb_v)
    pltpu.sync_copy(zb_v, acc.at[pl.ds(s * RPT, RPT)])

    plsc.subcore_barrier()

    @pl.loop(0, NCH)
    def _(k):
        pltpu.sync_copy(one_v, acc.at[dst_v.at[k]], add=True)

    plsc.subcore_barrier()
    pltpu.sync_copy(acc.at[pl.ds(s * RPT, RPT)],
                    out_hbm.at[c, pl.ds(s * RPT, RPT)])


def _dinv_of(dp_ref):
    deg = dp_ref[0] + dp_ref[1] + 1.0
    return lax.rsqrt(deg)


def _mm1_body(x_ref, w_ref, dp_ref, o_ref):
    h = lax.dot_general(x_ref[...], w_ref[...], (((1,), (1,)), ((), ())),
                        preferred_element_type=jnp.float32)
    o_ref[...] = h * _dinv_of(dp_ref)


def _comb1_body(p_ref, hs_ref, dp_ref, x_ref, b_ref, w_ref, h1_ref, hs2_ref):
    dinv = _dinv_of(dp_ref)
    agg = p_ref[0] + p_ref[1] + hs_ref[...]
    h1 = jnp.maximum(dinv * agg + b_ref[...] + x_ref[...], 0.0)
    h1_ref[...] = h1
    hs2_ref[...] = lax.dot_general(h1, w_ref[...], (((1,), (1,)), ((), ())),
                                   preferred_element_type=jnp.float32) * dinv


def _comb2_body(p_ref, hs_ref, dp_ref, h1_ref, b_ref, wh_ref, bh_ref, o_ref):
    dinv = _dinv_of(dp_ref)
    agg = p_ref[0] + p_ref[1] + hs_ref[...]
    h2 = jnp.maximum(dinv * agg + b_ref[...] + h1_ref[...], 0.0)
    o_ref[...] = lax.dot_general(h2, wh_ref[...], (((1,), (1,)), ((), ())),
                                 preferred_element_type=jnp.float32) + bh_ref[...]


def _row_spec():
    return pl.BlockSpec((128, D), lambda i: (i, 0))


_P_SPEC = pl.BlockSpec((NC, 128, D), lambda i: (0, i, 0))
_DP_SPEC = pl.BlockSpec((NC, 128, 1), lambda i: (0, i, 0))
_FULL_W = pl.BlockSpec((D, D), lambda i: (0, 0))
_FULL_B = pl.BlockSpec((1, D), lambda i: (0, 0))


def _mm1(x_pad, w, degp):
    return pl.pallas_call(
        _mm1_body,
        grid=(GRID,),
        in_specs=[_row_spec(), _FULL_W, _DP_SPEC],
        out_specs=_row_spec(),
        out_shape=jax.ShapeDtypeStruct((N_PAD, D), jnp.float32),
    )(x_pad, w, degp)


def _comb1(parts, hs1, degp, x_pad, b, w2):
    return pl.pallas_call(
        _comb1_body,
        grid=(GRID,),
        in_specs=[_P_SPEC, _row_spec(), _DP_SPEC, _row_spec(), _FULL_B, _FULL_W],
        out_specs=[_row_spec(), _row_spec()],
        out_shape=[jax.ShapeDtypeStruct((N_PAD, D), jnp.float32),
                   jax.ShapeDtypeStruct((N_PAD, D), jnp.float32)],
    )(parts, hs1, degp, x_pad, b, w2)


def _comb2(parts, hs2, degp, h1, b, wh, bh):
    return pl.pallas_call(
        _comb2_body,
        grid=(GRID,),
        in_specs=[_P_SPEC, _row_spec(), _DP_SPEC, _row_spec(), _FULL_B,
                  pl.BlockSpec((1, D), lambda i: (0, 0)),
                  pl.BlockSpec((1, 1), lambda i: (0, 0))],
        out_specs=pl.BlockSpec((128, 1), lambda i: (i, 0)),
        out_shape=jax.ShapeDtypeStruct((N_PAD, 1), jnp.float32),
    )(parts, hs2, degp, h1, b, wh, bh)


def kernel(x, edge_index, W1, b1, W2, b2, Wh, bh):
    x_pad = jnp.zeros((N_PAD, D), jnp.float32).at[:N].set(x)
    pad = E_PAD - E
    fill = jnp.full((pad,), N_PAD - 1, jnp.int32)
    src3 = jnp.concatenate([edge_index[0], fill]).reshape(NW, NCH, CH)
    dst3 = jnp.concatenate([edge_index[1], fill]).reshape(NW, NCH, CH)

    zer = jnp.zeros((CH, D), jnp.float32)
    one_col = jnp.ones((CH, 1), jnp.float32)
    zer_col = jnp.zeros((RPT, 1), jnp.float32)

    degp = _sc_degree(dst3, one_col, zer_col)

    b1r = b1.reshape(1, D)
    b2r = b2.reshape(1, D)
    bhr = bh.reshape(1, 1)

    hs1 = _mm1(x_pad, W1, degp)
    a1 = _sc_scatter(hs1, src3, dst3, zer)
    h1, hs2 = _comb1(a1, hs1, degp, x_pad, b1r, W2)
    a2 = _sc_scatter(hs2, src3, dst3, zer)
    out = _comb2(a2, hs2, degp, h1, b2r, Wh, bhr)
    return out[:N, 0]


# speedup vs baseline: 4.1705x; 4.1705x over previous
import functools

import jax
import jax.numpy as jnp
from jax import lax
from jax.experimental import pallas as pl
from jax.experimental.pallas import tpu as pltpu
from jax.experimental.pallas import tpu_sc as plsc

N = 10000
E = 320000
D = 128

NC = 2
NS = 16
NW = NC * NS
CH = 128
L = 16

NCH_D = -(-E // (NW * CH))
E_PAD_D = NW * NCH_D * CH

GSZ = 32
NG = 5
NCH_S = GSZ * NG
E_PAD_S = NS * NCH_S * CH

N_PAD = 10240
HALF = N_PAD // 2
ACC_R = HALF + 128
DUMP = HALF
ZPT = ACC_R // NS
DPT = N_PAD // NS
GRID = N_PAD // 128
NBLK = GRID // NC

_mesh = plsc.VectorSubcoreMesh(core_axis_name="c", subcore_axis_name="s")


@functools.partial(
    pl.kernel,
    out_type=jax.ShapeDtypeStruct((NC, ACC_R, D), jnp.float32),
    mesh=_mesh,
    scratch_types=[
        pltpu.VMEM((GSZ, CH), jnp.int32),
        pltpu.VMEM((GSZ, CH), jnp.int32),
        pltpu.VMEM((CH, D), jnp.float32),
        pltpu.VMEM_SHARED((ACC_R, D), jnp.float32),
        pltpu.SemaphoreType.DMA,
    ],
)
def _sc_scatter(hs_hbm, src_hbm, dst_hbm, zer_hbm, out_hbm,
                src_v, dst_v, rows_v, acc, gsem):
    c = lax.axis_index("c")
    s = lax.axis_index("s")
    base = c * HALF

    pltpu.sync_copy(zer_hbm, acc.at[pl.ds(s * ZPT, ZPT)])

    plsc.subcore_barrier()

    @pl.loop(0, NG)
    def _(g):
        pltpu.sync_copy(src_hbm.at[s, pl.ds(g * GSZ, GSZ)], src_v)
        pltpu.sync_copy(dst_hbm.at[s, pl.ds(g * GSZ, GSZ)], dst_v)

        @pl.loop(0, GSZ)
        def _(r):
            for j in range(CH // L):
                v = dst_v[r, pl.ds(j * L, L)] - base
                ok = (v >= 0) & (v < HALF)
                dst_v[r, pl.ds(j * L, L)] = jnp.where(ok, v, DUMP)

        @pl.loop(0, GSZ)
        def _(k):
            pltpu.async_copy(hs_hbm.at[src_v.at[k]], rows_v, gsem).wait()
            pltpu.sync_copy(rows_v, acc.at[dst_v.at[k]], add=True)

    plsc.subcore_barrier()

    pltpu.sync_copy(acc.at[pl.ds(s * ZPT, ZPT)],
                    out_hbm.at[c, pl.ds(s * ZPT, ZPT)])


EB = 2048
E_PAD_T = E_PAD_S
NCH_T = E_PAD_T // EB


def _deg_body(dst_ref, o_ref):
    @pl.when(pl.program_id(0) == 0)
    def _():
        o_ref[...] = jnp.zeros_like(o_ref)

    d = dst_ref[...]
    hi = d >> 7
    lo = d & 127
    uhi = (hi == lax.broadcasted_iota(jnp.int32, (1, GRID), 1)).astype(jnp.float32)
    ulo = (lo == lax.broadcasted_iota(jnp.int32, (1, D), 1)).astype(jnp.float32)
    o_ref[...] += lax.dot_general(uhi, ulo, (((0,), (0,)), ((), ())),
                                  preferred_element_type=jnp.float32)


def _deg_hist(dst_col):
    return pl.pallas_call(
        _deg_body,
        grid=(NCH_T,),
        in_specs=[pl.BlockSpec((EB, 1), lambda i: (i, 0))],
        out_specs=pl.BlockSpec((GRID, D), lambda i: (0, 0)),
        out_shape=jax.ShapeDtypeStruct((GRID, D), jnp.float32),
    )(dst_col)


def _dinv_of(dp_ref):
    deg = dp_ref[0] + 1.0
    return lax.rsqrt(deg)


def _mm1_body(x_ref, w_ref, dp_ref, o_ref):
    h = lax.dot_general(x_ref[...], w_ref[...], (((1,), (1,)), ((), ())),
                        preferred_element_type=jnp.float32)
    o_ref[...] = h * _dinv_of(dp_ref)


def _comb_body(a_ref, hs_ref, dp_ref, x_ref, b_ref, w_ref, h_ref, hsn_ref):
    dinv = _dinv_of(dp_ref)
    agg = a_ref[0] + hs_ref[...]
    h = jnp.maximum(dinv * agg + b_ref[...] + x_ref[...], 0.0)
    h_ref[...] = h
    hsn_ref[...] = lax.dot_general(h, w_ref[...], (((1,), (1,)), ((), ())),
                                   preferred_element_type=jnp.float32) * dinv


def _head_body(h_ref, wh_ref, bh_ref, o_ref):
    o_ref[...] = lax.dot_general(h_ref[...], wh_ref[...], (((1,), (0,)), ((), ())),
                                 preferred_element_type=jnp.float32) + bh_ref[0, 0]


def _row_spec():
    return pl.BlockSpec((128, D), lambda i: (i, 0))


_A_SPEC = pl.BlockSpec((1, 128, D), lambda i: (i // NBLK, i % NBLK, 0))
_DP_SPEC = pl.BlockSpec((1, 128, 1), lambda i: (i, 0, 0))
_FULL_W = pl.BlockSpec((D, D), lambda i: (0, 0))
_FULL_B = pl.BlockSpec((1, D), lambda i: (0, 0))


def _mm1(x_pad, w, degp):
    return pl.pallas_call(
        _mm1_body,
        grid=(GRID,),
        in_specs=[_row_spec(), _FULL_W, _DP_SPEC],
        out_specs=_row_spec(),
        out_shape=jax.ShapeDtypeStruct((N_PAD, D), jnp.float32),
    )(x_pad, w, degp)


def _comb(agg, hs, degp, resid, b, w_next):
    return pl.pallas_call(
        _comb_body,
        grid=(GRID,),
        in_specs=[_A_SPEC, _row_spec(), _DP_SPEC, _row_spec(), _FULL_B, _FULL_W],
        out_specs=[_row_spec(), _row_spec()],
        out_shape=[jax.ShapeDtypeStruct((N_PAD, D), jnp.float32),
                   jax.ShapeDtypeStruct((N_PAD, D), jnp.float32)],
    )(agg, hs, degp, resid, b, w_next)


def _head(h2, wh, bh):
    return pl.pallas_call(
        _head_body,
        grid=(GRID,),
        in_specs=[_row_spec(),
                  pl.BlockSpec((D, 1), lambda i: (0, 0)),
                  pl.BlockSpec((1, 1), lambda i: (0, 0))],
        out_specs=pl.BlockSpec((128, 1), lambda i: (i, 0)),
        out_shape=jax.ShapeDtypeStruct((N_PAD, 1), jnp.float32),
    )(h2, wh, bh)


def kernel(x, edge_index, W1, b1, W2, b2, Wh, bh):
    x_pad = jnp.zeros((N_PAD, D), jnp.float32).at[:N].set(x)

    fill_s = jnp.full((E_PAD_S - E,), N_PAD - 1, jnp.int32)
    src_s = jnp.concatenate([edge_index[0], fill_s]).reshape(NS, NCH_S, CH)
    dst_flat = jnp.concatenate([edge_index[1], fill_s])
    dst_s = dst_flat.reshape(NS, NCH_S, CH)

    zer = jnp.zeros((ZPT, D), jnp.float32)

    degp = _deg_hist(dst_flat.reshape(E_PAD_T, 1)).reshape(GRID, D, 1)

    bhr = bh.reshape(1, 1)

    hs1 = _mm1(x_pad, W1, degp)

    w_nexts = jnp.stack([W2, W2])
    bs = jnp.stack([b1.reshape(1, D), b2.reshape(1, D)])

    def _step(carry, xs):
        resid, hs = carry
        w_next, b = xs
        a = _sc_scatter(hs, src_s, dst_s, zer)
        h, hs_next = _comb(a, hs, degp, resid, b, w_next)
        return (h, hs_next), None

    (h2, _), _ = lax.scan(_step, (x_pad, hs1), (w_nexts, bs))
    out = _head(h2, Wh.reshape(1, D).T, bhr)
    return out[:N, 0]


# speedup vs baseline: 4.4247x; 1.0609x over previous
import functools

import jax
import jax.numpy as jnp
from jax import lax
from jax.experimental import pallas as pl
from jax.experimental.pallas import tpu as pltpu
from jax.experimental.pallas import tpu_sc as plsc

N = 10000
E = 320000
D = 128

NC = 2
NS = 16
NW = NC * NS
CH = 128
L = 16

NCH_D = -(-E // (NW * CH))
E_PAD_D = NW * NCH_D * CH

NCH_S = 160
E_PAD_S = NS * NCH_S * CH

N_PAD = 10240
HALF = N_PAD // 2
ACC_R = HALF + 128
DUMP = HALF
ZPT = ACC_R // NS
DPT = N_PAD // NS
GRID = N_PAD // 128
NBLK = GRID // NC

_mesh = plsc.VectorSubcoreMesh(core_axis_name="c", subcore_axis_name="s")


@functools.partial(
    pl.kernel,
    out_type=jax.ShapeDtypeStruct((NC, ACC_R, D), jnp.float32),
    mesh=_mesh,
    scratch_types=[
        pltpu.VMEM((NCH_S, CH), jnp.int32),
        pltpu.VMEM((NCH_S, CH), jnp.int32),
        pltpu.VMEM((CH, D), jnp.float32),
        pltpu.VMEM((CH, D), jnp.float32),
        pltpu.VMEM_SHARED((ACC_R, D), jnp.float32),
        pltpu.SemaphoreType.DMA,
    ],
)
def _sc_scatter(hs_hbm, src_hbm, dst_hbm, zer_hbm, out_hbm,
                src_v, dst_v, rows0_v, rows1_v, acc, gsem):
    c = lax.axis_index("c")
    s = lax.axis_index("s")
    base = c * HALF

    pltpu.sync_copy(src_hbm.at[s], src_v)
    pltpu.sync_copy(dst_hbm.at[s], dst_v)

    pltpu.sync_copy(zer_hbm, acc.at[pl.ds(s * ZPT, ZPT)])

    @pl.loop(0, NCH_S)
    def _(r):
        for j in range(CH // L):
            d = dst_v[r, pl.ds(j * L, L)]
            v = d - base
            ok = (v >= 0) & (v < HALF)
            dst_v[r, pl.ds(j * L, L)] = jnp.where(ok, v, DUMP + (d & 127))

    plsc.subcore_barrier()

    def _gather(k, buf):
        return pltpu.async_copy(hs_hbm.at[src_v.at[k]], buf, gsem)

    def _gather_wait(k, buf):
        pltpu.make_async_copy(hs_hbm.at[src_v.at[k]], buf, gsem).wait()

    def _scat(k, buf):
        pltpu.sync_copy(buf, acc.at[dst_v.at[k]], add=True)

    _gather(0, rows0_v)

    @pl.loop(0, NCH_S // 2 - 1)
    def _(i):
        k = 2 * i
        _gather_wait(k, rows0_v)
        _gather(k + 1, rows1_v)
        _scat(k, rows0_v)
        _gather_wait(k + 1, rows1_v)
        _gather(k + 2, rows0_v)
        _scat(k + 1, rows1_v)

    _gather_wait(NCH_S - 2, rows0_v)
    _gather(NCH_S - 1, rows1_v)
    _scat(NCH_S - 2, rows0_v)
    _gather_wait(NCH_S - 1, rows1_v)
    _scat(NCH_S - 1, rows1_v)

    plsc.subcore_barrier()

    pltpu.sync_copy(acc.at[pl.ds(s * ZPT, ZPT)],
                    out_hbm.at[c, pl.ds(s * ZPT, ZPT)])


EB = 2048
E_PAD_T = E_PAD_S
NCH_T = E_PAD_T // EB


def _deg_body(dst_ref, o_ref):
    @pl.when(pl.program_id(0) == 0)
    def _():
        o_ref[...] = jnp.zeros_like(o_ref)

    d = dst_ref[...]
    hi = d >> 7
    lo = d & 127
    uhi = (hi == lax.broadcasted_iota(jnp.int32, (1, GRID), 1)).astype(jnp.float32)
    ulo = (lo == lax.broadcasted_iota(jnp.int32, (1, D), 1)).astype(jnp.float32)
    o_ref[...] += lax.dot_general(uhi, ulo, (((0,), (0,)), ((), ())),
                                  preferred_element_type=jnp.float32)


def _deg_hist(dst_col):
    return pl.pallas_call(
        _deg_body,
        grid=(NCH_T,),
        in_specs=[pl.BlockSpec((EB, 1), lambda i: (i, 0))],
        out_specs=pl.BlockSpec((GRID, D), lambda i: (0, 0)),
        out_shape=jax.ShapeDtypeStruct((GRID, D), jnp.float32),
    )(dst_col)


def _dinv_of(dp_ref):
    deg = dp_ref[0] + 1.0
    return lax.rsqrt(deg)


def _mm1_body(x_ref, w_ref, dp_ref, o_ref):
    h = lax.dot_general(x_ref[...], w_ref[...], (((1,), (1,)), ((), ())),
                        preferred_element_type=jnp.float32)
    o_ref[...] = h * _dinv_of(dp_ref)


def _comb_body(a_ref, hs_ref, dp_ref, x_ref, b_ref, w_ref, h_ref, hsn_ref):
    dinv = _dinv_of(dp_ref)
    agg = a_ref[0] + hs_ref[...]
    h = jnp.maximum(dinv * agg + b_ref[...] + x_ref[...], 0.0)
    h_ref[...] = h
    hsn_ref[...] = lax.dot_general(h, w_ref[...], (((1,), (1,)), ((), ())),
                                   preferred_element_type=jnp.float32) * dinv


def _head_body(h_ref, wh_ref, bh_ref, o_ref):
    o_ref[...] = lax.dot_general(h_ref[...], wh_ref[...], (((1,), (0,)), ((), ())),
                                 preferred_element_type=jnp.float32) + bh_ref[0, 0]


def _row_spec():
    return pl.BlockSpec((128, D), lambda i: (i, 0))


_A_SPEC = pl.BlockSpec((1, 128, D), lambda i: (i // NBLK, i % NBLK, 0))
_DP_SPEC = pl.BlockSpec((1, 128, 1), lambda i: (i, 0, 0))
_FULL_W = pl.BlockSpec((D, D), lambda i: (0, 0))
_FULL_B = pl.BlockSpec((1, D), lambda i: (0, 0))


def _mm1(x_pad, w, degp):
    return pl.pallas_call(
        _mm1_body,
        grid=(GRID,),
        in_specs=[_row_spec(), _FULL_W, _DP_SPEC],
        out_specs=_row_spec(),
        out_shape=jax.ShapeDtypeStruct((N_PAD, D), jnp.float32),
    )(x_pad, w, degp)


def _comb(agg, hs, degp, resid, b, w_next):
    return pl.pallas_call(
        _comb_body,
        grid=(GRID,),
        in_specs=[_A_SPEC, _row_spec(), _DP_SPEC, _row_spec(), _FULL_B, _FULL_W],
        out_specs=[_row_spec(), _row_spec()],
        out_shape=[jax.ShapeDtypeStruct((N_PAD, D), jnp.float32),
                   jax.ShapeDtypeStruct((N_PAD, D), jnp.float32)],
    )(agg, hs, degp, resid, b, w_next)


def _head(h2, wh, bh):
    return pl.pallas_call(
        _head_body,
        grid=(GRID,),
        in_specs=[_row_spec(),
                  pl.BlockSpec((D, 1), lambda i: (0, 0)),
                  pl.BlockSpec((1, 1), lambda i: (0, 0))],
        out_specs=pl.BlockSpec((128, 1), lambda i: (i, 0)),
        out_shape=jax.ShapeDtypeStruct((N_PAD, 1), jnp.float32),
    )(h2, wh, bh)


def kernel(x, edge_index, W1, b1, W2, b2, Wh, bh):
    x_pad = jnp.zeros((N_PAD, D), jnp.float32).at[:N].set(x)

    fill_s = jnp.full((E_PAD_S - E,), N_PAD - 1, jnp.int32)
    src_s = jnp.concatenate([edge_index[0], fill_s]).reshape(NS, NCH_S, CH)
    dst_flat = jnp.concatenate([edge_index[1], fill_s])
    dst_s = dst_flat.reshape(NS, NCH_S, CH)

    zer = jnp.zeros((ZPT, D), jnp.float32)

    degp = _deg_hist(dst_flat.reshape(E_PAD_T, 1)).reshape(GRID, D, 1)

    bhr = bh.reshape(1, 1)

    hs1 = _mm1(x_pad, W1, degp)

    w_nexts = jnp.stack([W2, W2])
    bs = jnp.stack([b1.reshape(1, D), b2.reshape(1, D)])

    def _step(carry, xs):
        resid, hs = carry
        w_next, b = xs
        a = _sc_scatter(hs, src_s, dst_s, zer)
        h, hs_next = _comb(a, hs, degp, resid, b, w_next)
        return (h, hs_next), None

    (h2, _), _ = lax.scan(_step, (x_pad, hs1), (w_nexts, bs))
    out = _head(h2, Wh.reshape(1, D).T, bhr)
    return out[:N, 0]


# speedup vs baseline: 6.1828x; 1.3974x over previous
import functools

import jax
import jax.numpy as jnp
from jax import lax
from jax.experimental import pallas as pl
from jax.experimental.pallas import tpu as pltpu
from jax.experimental.pallas import tpu_sc as plsc

N = 10000
E = 320000
D = 128

NC = 2
NS = 16
NW = NC * NS
CH = 128
L = 16

NCH_D = -(-E // (NW * CH))
E_PAD_D = NW * NCH_D * CH

GSZ = 16
NG = 5
NCH_W = GSZ * NG
E_PAD_S = NW * NCH_W * CH

N_PAD = 10240
RPT = N_PAD // NS
GRID = N_PAD // 128

_mesh = plsc.VectorSubcoreMesh(core_axis_name="c", subcore_axis_name="s")


@functools.partial(
    pl.kernel,
    out_type=jax.ShapeDtypeStruct((NC, N_PAD, D), jnp.float32),
    mesh=_mesh,
    scratch_types=[
        pltpu.VMEM((GSZ, CH), jnp.int32),
        pltpu.VMEM((GSZ, CH), jnp.int32),
        pltpu.VMEM((CH, D), jnp.float32),
        pltpu.VMEM((CH, D), jnp.float32),
        pltpu.VMEM_SHARED((N_PAD, D), jnp.float32),
        pltpu.SemaphoreType.DMA,
    ],
)
def _sc_scatter(hs_hbm, src_hbm, dst_hbm, zer_hbm, out_hbm,
                src_v, dst_v, rows0_v, rows1_v, acc, gsem):
    c = lax.axis_index("c")
    s = lax.axis_index("s")
    w = c * NS + s

    pltpu.sync_copy(zer_hbm, acc.at[pl.ds(s * RPT, RPT)])
    plsc.subcore_barrier()

    def _gather(k, buf):
        return pltpu.async_copy(hs_hbm.at[src_v.at[k]], buf, gsem)

    def _gather_wait(k, buf):
        pltpu.make_async_copy(hs_hbm.at[src_v.at[k]], buf, gsem).wait()

    def _scat(k, buf):
        pltpu.sync_copy(buf, acc.at[dst_v.at[k]], add=True)

    @pl.loop(0, NG)
    def _(g):
        pltpu.sync_copy(src_hbm.at[w, pl.ds(g * GSZ, GSZ)], src_v)
        pltpu.sync_copy(dst_hbm.at[w, pl.ds(g * GSZ, GSZ)], dst_v)

        _gather(0, rows0_v)

        @pl.loop(0, GSZ // 2 - 1)
        def _(i):
            k = 2 * i
            _gather_wait(k, rows0_v)
            _gather(k + 1, rows1_v)
            _scat(k, rows0_v)
            _gather_wait(k + 1, rows1_v)
            _gather(k + 2, rows0_v)
            _scat(k + 1, rows1_v)

        _gather_wait(GSZ - 2, rows0_v)
        _gather(GSZ - 1, rows1_v)
        _scat(GSZ - 2, rows0_v)
        _gather_wait(GSZ - 1, rows1_v)
        _scat(GSZ - 1, rows1_v)

    plsc.subcore_barrier()

    pltpu.sync_copy(acc.at[pl.ds(s * RPT, RPT)],
                    out_hbm.at[c, pl.ds(s * RPT, RPT)])


EB = 2048
E_PAD_T = E_PAD_S
NCH_T = E_PAD_T // EB


def _deg_body(dst_ref, o_ref):
    @pl.when(pl.program_id(0) == 0)
    def _():
        o_ref[...] = jnp.zeros_like(o_ref)

    d = dst_ref[...]
    hi = d >> 7
    lo = d & 127
    uhi = (hi == lax.broadcasted_iota(jnp.int32, (1, GRID), 1)).astype(jnp.float32)
    ulo = (lo == lax.broadcasted_iota(jnp.int32, (1, D), 1)).astype(jnp.float32)
    o_ref[...] += lax.dot_general(uhi, ulo, (((0,), (0,)), ((), ())),
                                  preferred_element_type=jnp.float32)


def _deg_hist(dst_col):
    return pl.pallas_call(
        _deg_body,
        grid=(NCH_T,),
        in_specs=[pl.BlockSpec((EB, 1), lambda i: (i, 0))],
        out_specs=pl.BlockSpec((GRID, D), lambda i: (0, 0)),
        out_shape=jax.ShapeDtypeStruct((GRID, D), jnp.float32),
    )(dst_col)


def _dinv_of(dp_ref):
    deg = dp_ref[0] + 1.0
    return lax.rsqrt(deg)


def _mm1_body(x_ref, w_ref, dp_ref, o_ref):
    h = lax.dot_general(x_ref[...], w_ref[...], (((1,), (1,)), ((), ())),
                        preferred_element_type=jnp.float32)
    o_ref[...] = h * _dinv_of(dp_ref)


def _comb_body(a_ref, hs_ref, dp_ref, x_ref, b_ref, w_ref, h_ref, hsn_ref):
    dinv = _dinv_of(dp_ref)
    agg = a_ref[0] + a_ref[1] + hs_ref[...]
    h = jnp.maximum(dinv * agg + b_ref[...] + x_ref[...], 0.0)
    h_ref[...] = h
    hsn_ref[...] = lax.dot_general(h, w_ref[...], (((1,), (1,)), ((), ())),
                                   preferred_element_type=jnp.float32) * dinv


def _head_body(h_ref, wh_ref, bh_ref, o_ref):
    o_ref[...] = lax.dot_general(h_ref[...], wh_ref[...], (((1,), (0,)), ((), ())),
                                 preferred_element_type=jnp.float32) + bh_ref[0, 0]


def _row_spec():
    return pl.BlockSpec((128, D), lambda i: (i, 0))


_A_SPEC = pl.BlockSpec((NC, 128, D), lambda i: (0, i, 0))
_DP_SPEC = pl.BlockSpec((1, 128, 1), lambda i: (i, 0, 0))
_FULL_W = pl.BlockSpec((D, D), lambda i: (0, 0))
_FULL_B = pl.BlockSpec((1, D), lambda i: (0, 0))


def _mm1(x_pad, w, degp):
    return pl.pallas_call(
        _mm1_body,
        grid=(GRID,),
        in_specs=[_row_spec(), _FULL_W, _DP_SPEC],
        out_specs=_row_spec(),
        out_shape=jax.ShapeDtypeStruct((N_PAD, D), jnp.float32),
    )(x_pad, w, degp)


def _comb(agg, hs, degp, resid, b, w_next):
    return pl.pallas_call(
        _comb_body,
        grid=(GRID,),
        in_specs=[_A_SPEC, _row_spec(), _DP_SPEC, _row_spec(), _FULL_B, _FULL_W],
        out_specs=[_row_spec(), _row_spec()],
        out_shape=[jax.ShapeDtypeStruct((N_PAD, D), jnp.float32),
                   jax.ShapeDtypeStruct((N_PAD, D), jnp.float32)],
    )(agg, hs, degp, resid, b, w_next)


def _head(h2, wh, bh):
    return pl.pallas_call(
        _head_body,
        grid=(GRID,),
        in_specs=[_row_spec(),
                  pl.BlockSpec((D, 1), lambda i: (0, 0)),
                  pl.BlockSpec((1, 1), lambda i: (0, 0))],
        out_specs=pl.BlockSpec((128, 1), lambda i: (i, 0)),
        out_shape=jax.ShapeDtypeStruct((N_PAD, 1), jnp.float32),
    )(h2, wh, bh)


def kernel(x, edge_index, W1, b1, W2, b2, Wh, bh):
    x_pad = jnp.zeros((N_PAD, D), jnp.float32).at[:N].set(x)

    fill_s = jnp.full((E_PAD_S - E,), N_PAD - 1, jnp.int32)
    src_s = jnp.concatenate([edge_index[0], fill_s]).reshape(NW, NCH_W, CH)
    dst_flat = jnp.concatenate([edge_index[1], fill_s])
    dst_s = dst_flat.reshape(NW, NCH_W, CH)

    zer = jnp.zeros((RPT, D), jnp.float32)

    degp = _deg_hist(dst_flat.reshape(E_PAD_T, 1)).reshape(GRID, D, 1)

    bhr = bh.reshape(1, 1)

    hs1 = _mm1(x_pad, W1, degp)

    w_nexts = jnp.stack([W2, W2])
    bs = jnp.stack([b1.reshape(1, D), b2.reshape(1, D)])

    def _step(carry, xs):
        resid, hs = carry
        w_next, b = xs
        a = _sc_scatter(hs, src_s, dst_s, zer)
        h, hs_next = _comb(a, hs, degp, resid, b, w_next)
        return (h, hs_next), None

    (h2, _), _ = lax.scan(_step, (x_pad, hs1), (w_nexts, bs))
    out = _head(h2, Wh.reshape(1, D).T, bhr)
    return out[:N, 0]


# speedup vs baseline: 6.3511x; 1.0272x over previous
import functools

import jax
import jax.numpy as jnp
from jax import lax
from jax.experimental import pallas as pl
from jax.experimental.pallas import tpu as pltpu
from jax.experimental.pallas import tpu_sc as plsc

N = 10000
E = 320000
D = 128

NC = 2
NS = 16
NW = NC * NS
CH = 128
L = 16

NCH_D = -(-E // (NW * CH))
E_PAD_D = NW * NCH_D * CH

GSZ = 16
NG = 5
NCH_W = GSZ * NG
E_PAD_S = NW * NCH_W * CH

N_PAD = 10240
RPT = N_PAD // NS
GRID = N_PAD // 128

_mesh = plsc.VectorSubcoreMesh(core_axis_name="c", subcore_axis_name="s")


@functools.partial(
    pl.kernel,
    out_type=jax.ShapeDtypeStruct((NC, N_PAD, D), jnp.float32),
    mesh=_mesh,
    scratch_types=[
        pltpu.VMEM((GSZ, CH), jnp.int32),
        pltpu.VMEM((GSZ, CH), jnp.int32),
        pltpu.VMEM((CH, D), jnp.float32),
        pltpu.VMEM((CH, D), jnp.float32),
        pltpu.VMEM_SHARED((N_PAD, D), jnp.float32),
        pltpu.SemaphoreType.DMA,
    ],
)
def _sc_scatter(hs_hbm, src_hbm, dst_hbm, zer_hbm, out_hbm,
                src_v, dst_v, rows0_v, rows1_v, acc, gsem):
    c = lax.axis_index("c")
    s = lax.axis_index("s")
    w = c * NS + s

    pltpu.sync_copy(zer_hbm, acc.at[pl.ds(s * RPT, RPT)])
    plsc.subcore_barrier()

    def _gather(k, buf):
        return pltpu.async_copy(hs_hbm.at[src_v.at[k]], buf, gsem)

    def _gather_wait(k, buf):
        pltpu.make_async_copy(hs_hbm.at[src_v.at[k]], buf, gsem).wait()

    def _scat(k, buf):
        pltpu.sync_copy(buf, acc.at[dst_v.at[k]], add=True)

    @pl.loop(0, NG)
    def _(g):
        pltpu.sync_copy(src_hbm.at[w, pl.ds(g * GSZ, GSZ)], src_v)
        pltpu.sync_copy(dst_hbm.at[w, pl.ds(g * GSZ, GSZ)], dst_v)

        _gather(0, rows0_v)

        @pl.loop(0, GSZ // 2 - 1)
        def _(i):
            k = 2 * i
            _gather_wait(k, rows0_v)
            _gather(k + 1, rows1_v)
            _scat(k, rows0_v)
            _gather_wait(k + 1, rows1_v)
            _gather(k + 2, rows0_v)
            _scat(k + 1, rows1_v)

        _gather_wait(GSZ - 2, rows0_v)
        _gather(GSZ - 1, rows1_v)
        _scat(GSZ - 2, rows0_v)
        _gather_wait(GSZ - 1, rows1_v)
        _scat(GSZ - 1, rows1_v)

    plsc.subcore_barrier()

    pltpu.sync_copy(acc.at[pl.ds(s * RPT, RPT)],
                    out_hbm.at[c, pl.ds(s * RPT, RPT)])


EB = 4096
E_PAD_T = E_PAD_S
NCH_T = E_PAD_T // EB


def _deg_body(dst_ref, o_ref):
    @pl.when(pl.program_id(0) == 0)
    def _():
        o_ref[...] = jnp.zeros_like(o_ref)

    d = dst_ref[...]
    hi = d >> 7
    lo = d & 127
    uhi = (hi == lax.broadcasted_iota(jnp.int32, (1, GRID), 1)).astype(jnp.bfloat16)
    ulo = (lo == lax.broadcasted_iota(jnp.int32, (1, D), 1)).astype(jnp.bfloat16)
    o_ref[...] += lax.dot_general(uhi, ulo, (((0,), (0,)), ((), ())),
                                  preferred_element_type=jnp.float32)


def _deg_hist(dst_col):
    return pl.pallas_call(
        _deg_body,
        grid=(NCH_T,),
        in_specs=[pl.BlockSpec((EB, 1), lambda i: (i, 0))],
        out_specs=pl.BlockSpec((GRID, D), lambda i: (0, 0)),
        out_shape=jax.ShapeDtypeStruct((GRID, D), jnp.float32),
    )(dst_col)


def _dinv_of(dp_ref):
    deg = dp_ref[0] + 1.0
    return lax.rsqrt(deg)


def _mm1_body(x_ref, w_ref, dp_ref, o_ref):
    h = lax.dot_general(x_ref[...], w_ref[...], (((1,), (1,)), ((), ())),
                        preferred_element_type=jnp.float32)
    o_ref[...] = h * _dinv_of(dp_ref)


def _comb_body(a_ref, hs_ref, dp_ref, x_ref, b_ref, w_ref, h_ref, hsn_ref):
    dinv = _dinv_of(dp_ref)
    agg = a_ref[0] + a_ref[1] + hs_ref[...]
    h = jnp.maximum(dinv * agg + b_ref[...] + x_ref[...], 0.0)
    h_ref[...] = h
    hsn_ref[...] = lax.dot_general(h, w_ref[...], (((1,), (1,)), ((), ())),
                                   preferred_element_type=jnp.float32) * dinv


def _head_body(h_ref, wh_ref, bh_ref, o_ref):
    o_ref[...] = lax.dot_general(h_ref[...], wh_ref[...], (((1,), (0,)), ((), ())),
                                 preferred_element_type=jnp.float32) + bh_ref[0, 0]


def _row_spec():
    return pl.BlockSpec((128, D), lambda i: (i, 0))


_A_SPEC = pl.BlockSpec((NC, 128, D), lambda i: (0, i, 0))
_DP_SPEC = pl.BlockSpec((1, 128, 1), lambda i: (i, 0, 0))
_FULL_W = pl.BlockSpec((D, D), lambda i: (0, 0))
_FULL_B = pl.BlockSpec((1, D), lambda i: (0, 0))


def _mm1(x_pad, w, degp):
    return pl.pallas_call(
        _mm1_body,
        grid=(GRID,),
        in_specs=[_row_spec(), _FULL_W, _DP_SPEC],
        out_specs=_row_spec(),
        out_shape=jax.ShapeDtypeStruct((N_PAD, D), jnp.float32),
    )(x_pad, w, degp)


def _comb(agg, hs, degp, resid, b, w_next):
    return pl.pallas_call(
        _comb_body,
        grid=(GRID,),
        in_specs=[_A_SPEC, _row_spec(), _DP_SPEC, _row_spec(), _FULL_B, _FULL_W],
        out_specs=[_row_spec(), _row_spec()],
        out_shape=[jax.ShapeDtypeStruct((N_PAD, D), jnp.float32),
                   jax.ShapeDtypeStruct((N_PAD, D), jnp.float32)],
    )(agg, hs, degp, resid, b, w_next)


def _head(h2, wh, bh):
    return pl.pallas_call(
        _head_body,
        grid=(GRID,),
        in_specs=[_row_spec(),
                  pl.BlockSpec((D, 1), lambda i: (0, 0)),
                  pl.BlockSpec((1, 1), lambda i: (0, 0))],
        out_specs=pl.BlockSpec((128, 1), lambda i: (i, 0)),
        out_shape=jax.ShapeDtypeStruct((N_PAD, 1), jnp.float32),
    )(h2, wh, bh)


def kernel(x, edge_index, W1, b1, W2, b2, Wh, bh):
    x_pad = jnp.zeros((N_PAD, D), jnp.float32).at[:N].set(x)

    fill_s = jnp.full((E_PAD_S - E,), N_PAD - 1, jnp.int32)
    src_s = jnp.concatenate([edge_index[0], fill_s]).reshape(NW, NCH_W, CH)
    dst_flat = jnp.concatenate([edge_index[1], fill_s])
    dst_s = dst_flat.reshape(NW, NCH_W, CH)

    zer = jnp.zeros((RPT, D), jnp.float32)

    degp = _deg_hist(dst_flat.reshape(E_PAD_T, 1)).reshape(GRID, D, 1)

    bhr = bh.reshape(1, 1)

    hs1 = _mm1(x_pad, W1, degp)

    w_nexts = jnp.stack([W2, W2])
    bs = jnp.stack([b1.reshape(1, D), b2.reshape(1, D)])

    def _step(carry, xs):
        resid, hs = carry
        w_next, b = xs
        a = _sc_scatter(hs, src_s, dst_s, zer)
        h, hs_next = _comb(a, hs, degp, resid, b, w_next)
        return (h, hs_next), None

    (h2, _), _ = lax.scan(_step, (x_pad, hs1), (w_nexts, bs))
    out = _head(h2, Wh.reshape(1, D).T, bhr)
    return out[:N, 0]


# speedup vs baseline: 12.3782x; 1.9490x over previous
import functools

import jax
import jax.numpy as jnp
from jax import lax
from jax.experimental import pallas as pl
from jax.experimental.pallas import tpu as pltpu
from jax.experimental.pallas import tpu_sc as plsc

N = 10000
E = 320000
D = 128

NC = 2
NS = 16
NW = NC * NS
CH = 128
L = 16

NCH_D = -(-E // (NW * CH))
E_PAD_D = NW * NCH_D * CH

GSZ = 16
NG = 5
NCH_W = GSZ * NG
E_PAD_S = NW * NCH_W * CH

N_PAD = 10240
RPT = N_PAD // NS
GRID = N_PAD // 128

_mesh = plsc.VectorSubcoreMesh(core_axis_name="c", subcore_axis_name="s")


@functools.partial(
    pl.kernel,
    out_type=jax.ShapeDtypeStruct((NC, N_PAD, D), jnp.float32),
    mesh=_mesh,
    scratch_types=[
        pltpu.VMEM((GSZ, CH), jnp.int32),
        pltpu.VMEM((GSZ, CH), jnp.int32),
        pltpu.VMEM((CH, D), jnp.float32),
        pltpu.VMEM((CH, D), jnp.float32),
        pltpu.VMEM_SHARED((N_PAD, D), jnp.float32),
        pltpu.SemaphoreType.DMA,
    ],
)
def _sc_scatter(hs_hbm, src_hbm, dst_hbm, zer_hbm, out_hbm,
                src_v, dst_v, rows0_v, rows1_v, acc, gsem):
    c = lax.axis_index("c")
    s = lax.axis_index("s")
    w = c * NS + s

    pltpu.sync_copy(zer_hbm, acc.at[pl.ds(s * RPT, RPT)])
    plsc.subcore_barrier()

    def _gather(k, buf):
        return pltpu.async_copy(hs_hbm.at[src_v.at[k]], buf, gsem)

    def _gather_wait(k, buf):
        pltpu.make_async_copy(hs_hbm.at[src_v.at[k]], buf, gsem).wait()

    def _scat(k, buf):
        pltpu.sync_copy(buf, acc.at[dst_v.at[k]], add=True)

    @pl.loop(0, NG)
    def _(g):
        pltpu.sync_copy(src_hbm.at[w, pl.ds(g * GSZ, GSZ)], src_v)
        pltpu.sync_copy(dst_hbm.at[w, pl.ds(g * GSZ, GSZ)], dst_v)

        _gather(0, rows0_v)

        @pl.loop(0, GSZ // 2 - 1)
        def _(i):
            k = 2 * i
            _gather_wait(k, rows0_v)
            _gather(k + 1, rows1_v)
            _scat(k, rows0_v)
            _gather_wait(k + 1, rows1_v)
            _gather(k + 2, rows0_v)
            _scat(k + 1, rows1_v)

        _gather_wait(GSZ - 2, rows0_v)
        _gather(GSZ - 1, rows1_v)
        _scat(GSZ - 2, rows0_v)
        _gather_wait(GSZ - 1, rows1_v)
        _scat(GSZ - 1, rows1_v)

    plsc.subcore_barrier()

    pltpu.sync_copy(acc.at[pl.ds(s * RPT, RPT)],
                    out_hbm.at[c, pl.ds(s * RPT, RPT)])


EB = 4096
E_PAD_T = E_PAD_S
NCH_T = E_PAD_T // EB


def _deg_body(dst_ref, o_ref):
    @pl.when(pl.program_id(0) == 0)
    def _():
        o_ref[...] = jnp.zeros_like(o_ref)

    d = dst_ref[...]
    hi = d >> 7
    lo = d & 127
    uhi = (hi == lax.broadcasted_iota(jnp.int32, (1, GRID), 1)).astype(jnp.bfloat16)
    ulo = (lo == lax.broadcasted_iota(jnp.int32, (1, D), 1)).astype(jnp.bfloat16)
    o_ref[...] += lax.dot_general(uhi, ulo, (((0,), (0,)), ((), ())),
                                  preferred_element_type=jnp.float32)


def _deg_hist(dst_col):
    return pl.pallas_call(
        _deg_body,
        grid=(NCH_T,),
        in_specs=[pl.BlockSpec((EB, 1), lambda i: (i, 0))],
        out_specs=pl.BlockSpec((GRID, D), lambda i: (0, 0)),
        out_shape=jax.ShapeDtypeStruct((GRID, D), jnp.float32),
    )(dst_col)


def _dinv_of(dp_ref):
    deg = dp_ref[0] + 1.0
    return lax.rsqrt(deg)


def _mm1_body(x_ref, w_ref, dp_ref, o_ref):
    h = lax.dot_general(x_ref[...], w_ref[...], (((1,), (1,)), ((), ())),
                        preferred_element_type=jnp.float32)
    o_ref[...] = h * _dinv_of(dp_ref)


def _comb_body(a_ref, hs_ref, dp_ref, x_ref, b_ref, w_ref, h_ref, hsn_ref):
    dinv = _dinv_of(dp_ref)
    agg = a_ref[0] + a_ref[1] + hs_ref[...]
    h = jnp.maximum(dinv * agg + b_ref[...] + x_ref[...], 0.0)
    h_ref[...] = h
    hsn_ref[...] = lax.dot_general(h, w_ref[...], (((1,), (1,)), ((), ())),
                                   preferred_element_type=jnp.float32) * dinv


def _head_body(h_ref, wh_ref, bh_ref, o_ref):
    o_ref[...] = lax.dot_general(h_ref[...], wh_ref[...], (((1,), (0,)), ((), ())),
                                 preferred_element_type=jnp.float32) + bh_ref[0, 0]


def _row_spec():
    return pl.BlockSpec((128, D), lambda i: (i, 0))


_A_SPEC = pl.BlockSpec((NC, 128, D), lambda i: (0, i, 0))
_DP_SPEC = pl.BlockSpec((1, 128, 1), lambda i: (i, 0, 0))
_FULL_W = pl.BlockSpec((D, D), lambda i: (0, 0))
_FULL_B = pl.BlockSpec((1, D), lambda i: (0, 0))


def _mm1(x_pad, w, degp):
    return pl.pallas_call(
        _mm1_body,
        grid=(GRID,),
        in_specs=[_row_spec(), _FULL_W, _DP_SPEC],
        out_specs=_row_spec(),
        out_shape=jax.ShapeDtypeStruct((N_PAD, D), jnp.float32),
    )(x_pad, w, degp)


def _comb(agg, hs, degp, resid, b, w_next):
    return pl.pallas_call(
        _comb_body,
        grid=(GRID,),
        in_specs=[_A_SPEC, _row_spec(), _DP_SPEC, _row_spec(), _FULL_B, _FULL_W],
        out_specs=[_row_spec(), _row_spec()],
        out_shape=[jax.ShapeDtypeStruct((N_PAD, D), jnp.float32),
                   jax.ShapeDtypeStruct((N_PAD, D), jnp.float32)],
    )(agg, hs, degp, resid, b, w_next)


def _head(h2, wh, bh):
    return pl.pallas_call(
        _head_body,
        grid=(GRID,),
        in_specs=[_row_spec(),
                  pl.BlockSpec((D, 1), lambda i: (0, 0)),
                  pl.BlockSpec((1, 1), lambda i: (0, 0))],
        out_specs=pl.BlockSpec((128, 1), lambda i: (i, 0)),
        out_shape=jax.ShapeDtypeStruct((N_PAD, 1), jnp.float32),
    )(h2, wh, bh)


def kernel(x, edge_index, W1, b1, W2, b2, Wh, bh):
    x_pad = jnp.zeros((N_PAD, D), jnp.float32).at[:N].set(x)

    fill_s = (N + jnp.arange(E_PAD_S - E, dtype=jnp.int32) % (N_PAD - N))
    src_s = jnp.concatenate([edge_index[0], fill_s]).reshape(NW, NCH_W, CH)
    dst_flat = jnp.concatenate([edge_index[1], fill_s])
    dst_s = dst_flat.reshape(NW, NCH_W, CH)

    zer = jnp.zeros((RPT, D), jnp.float32)

    degp = _deg_hist(dst_flat.reshape(E_PAD_T, 1)).reshape(GRID, D, 1)

    bhr = bh.reshape(1, 1)

    hs1 = _mm1(x_pad, W1, degp)

    w_nexts = jnp.stack([W2, W2])
    bs = jnp.stack([b1.reshape(1, D), b2.reshape(1, D)])

    def _step(carry, xs):
        resid, hs = carry
        w_next, b = xs
        a = _sc_scatter(hs, src_s, dst_s, zer)
        h, hs_next = _comb(a, hs, degp, resid, b, w_next)
        return (h, hs_next), None

    (h2, _), _ = lax.scan(_step, (x_pad, hs1), (w_nexts, bs))
    out = _head(h2, Wh.reshape(1, D).T, bhr)
    return out[:N, 0]


# speedup vs baseline: 15.7561x; 1.2729x over previous
import functools

import jax
import jax.numpy as jnp
from jax import lax
from jax.experimental import pallas as pl
from jax.experimental.pallas import tpu as pltpu
from jax.experimental.pallas import tpu_sc as plsc

N = 10000
E = 320000
D = 128

NC = 2
NS = 16
NW = NC * NS
CH = 128
L = 16

NCH_D = -(-E // (NW * CH))
E_PAD_D = NW * NCH_D * CH

GSZ = 16
NG = 5
NCH_W = GSZ * NG
E_PAD_S = NW * NCH_W * CH

N_PAD = 10240
RPT = N_PAD // NS
GRID = N_PAD // 128

_mesh = plsc.VectorSubcoreMesh(core_axis_name="c", subcore_axis_name="s")


@functools.partial(
    pl.kernel,
    out_type=jax.ShapeDtypeStruct((NC, N_PAD, D), jnp.float32),
    mesh=_mesh,
    scratch_types=[
        pltpu.VMEM((GSZ, CH), jnp.int32),
        pltpu.VMEM((GSZ, CH), jnp.int32),
        pltpu.VMEM((CH, D), jnp.float32),
        pltpu.VMEM((CH, D), jnp.float32),
        pltpu.VMEM_SHARED((N_PAD, D), jnp.float32),
        pltpu.SemaphoreType.DMA,
    ],
)
def _sc_scatter(hs_hbm, src_hbm, dst_hbm, zer_hbm, out_hbm,
                src_v, dst_v, rows0_v, rows1_v, acc, gsem):
    c = lax.axis_index("c")
    s = lax.axis_index("s")
    w = c * NS + s

    pltpu.sync_copy(zer_hbm, acc.at[pl.ds(s * RPT, RPT)])
    plsc.subcore_barrier()

    def _gather(k, buf):
        return pltpu.async_copy(hs_hbm.at[src_v.at[k]], buf, gsem)

    def _gather_wait(k, buf):
        pltpu.make_async_copy(hs_hbm.at[src_v.at[k]], buf, gsem).wait()

    def _scat(k, buf):
        pltpu.sync_copy(buf, acc.at[dst_v.at[k]], add=True)

    @pl.loop(0, NG)
    def _(g):
        pltpu.sync_copy(src_hbm.at[w, pl.ds(g * GSZ, GSZ)], src_v)
        pltpu.sync_copy(dst_hbm.at[w, pl.ds(g * GSZ, GSZ)], dst_v)

        _gather(0, rows0_v)

        @pl.loop(0, GSZ // 2 - 1)
        def _(i):
            k = 2 * i
            _gather_wait(k, rows0_v)
            _gather(k + 1, rows1_v)
            _scat(k, rows0_v)
            _gather_wait(k + 1, rows1_v)
            _gather(k + 2, rows0_v)
            _scat(k + 1, rows1_v)

        _gather_wait(GSZ - 2, rows0_v)
        _gather(GSZ - 1, rows1_v)
        _scat(GSZ - 2, rows0_v)
        _gather_wait(GSZ - 1, rows1_v)
        _scat(GSZ - 1, rows1_v)

    plsc.subcore_barrier()

    pltpu.sync_copy(acc.at[pl.ds(s * RPT, RPT)],
                    out_hbm.at[c, pl.ds(s * RPT, RPT)])


EB = 4096
E_PAD_T = E_PAD_S
NCH_T = E_PAD_T // EB


def _deg_body(dst_ref, o_ref):
    @pl.when(pl.program_id(0) == 0)
    def _():
        o_ref[...] = jnp.zeros_like(o_ref)

    d = dst_ref[...]
    hi = d >> 7
    lo = d & 127
    uhiT = (lax.broadcasted_iota(jnp.int32, (GRID, EB), 0) == hi).astype(jnp.bfloat16)
    uloT = (lax.broadcasted_iota(jnp.int32, (D, EB), 0) == lo).astype(jnp.bfloat16)
    o_ref[...] += lax.dot_general(uhiT, uloT, (((1,), (1,)), ((), ())),
                                  preferred_element_type=jnp.float32)


def _deg_hist(dst_row):
    return pl.pallas_call(
        _deg_body,
        grid=(NCH_T,),
        in_specs=[pl.BlockSpec((1, EB), lambda i: (0, i))],
        out_specs=pl.BlockSpec((GRID, D), lambda i: (0, 0)),
        out_shape=jax.ShapeDtypeStruct((GRID, D), jnp.float32),
    )(dst_row)


def _dinv_of(dp_ref):
    deg = dp_ref[0] + 1.0
    return lax.rsqrt(deg)


def _mm1_body(x_ref, w_ref, dp_ref, o_ref):
    h = lax.dot_general(x_ref[...], w_ref[...], (((1,), (1,)), ((), ())),
                        preferred_element_type=jnp.float32)
    o_ref[...] = h * _dinv_of(dp_ref)


def _comb_body(a_ref, hs_ref, dp_ref, x_ref, b_ref, w_ref, h_ref, hsn_ref):
    dinv = _dinv_of(dp_ref)
    agg = a_ref[0] + a_ref[1] + hs_ref[...]
    h = jnp.maximum(dinv * agg + b_ref[...] + x_ref[...], 0.0)
    h_ref[...] = h
    hsn_ref[...] = lax.dot_general(h, w_ref[...], (((1,), (1,)), ((), ())),
                                   preferred_element_type=jnp.float32) * dinv


def _head_body(h_ref, wh_ref, bh_ref, o_ref):
    o_ref[...] = lax.dot_general(h_ref[...], wh_ref[...], (((1,), (0,)), ((), ())),
                                 preferred_element_type=jnp.float32) + bh_ref[0, 0]


def _row_spec():
    return pl.BlockSpec((128, D), lambda i: (i, 0))


_A_SPEC = pl.BlockSpec((NC, 128, D), lambda i: (0, i, 0))
_DP_SPEC = pl.BlockSpec((1, 128, 1), lambda i: (i, 0, 0))
_FULL_W = pl.BlockSpec((D, D), lambda i: (0, 0))
_FULL_B = pl.BlockSpec((1, D), lambda i: (0, 0))


def _mm1(x_pad, w, degp):
    return pl.pallas_call(
        _mm1_body,
        grid=(GRID,),
        in_specs=[_row_spec(), _FULL_W, _DP_SPEC],
        out_specs=_row_spec(),
        out_shape=jax.ShapeDtypeStruct((N_PAD, D), jnp.float32),
    )(x_pad, w, degp)


def _comb(agg, hs, degp, resid, b, w_next):
    return pl.pallas_call(
        _comb_body,
        grid=(GRID,),
        in_specs=[_A_SPEC, _row_spec(), _DP_SPEC, _row_spec(), _FULL_B, _FULL_W],
        out_specs=[_row_spec(), _row_spec()],
        out_shape=[jax.ShapeDtypeStruct((N_PAD, D), jnp.float32),
                   jax.ShapeDtypeStruct((N_PAD, D), jnp.float32)],
    )(agg, hs, degp, resid, b, w_next)


def _head(h2, wh, bh):
    return pl.pallas_call(
        _head_body,
        grid=(GRID,),
        in_specs=[_row_spec(),
                  pl.BlockSpec((D, 1), lambda i: (0, 0)),
                  pl.BlockSpec((1, 1), lambda i: (0, 0))],
        out_specs=pl.BlockSpec((128, 1), lambda i: (i, 0)),
        out_shape=jax.ShapeDtypeStruct((N_PAD, 1), jnp.float32),
    )(h2, wh, bh)


def kernel(x, edge_index, W1, b1, W2, b2, Wh, bh):
    x_pad = jnp.zeros((N_PAD, D), jnp.float32).at[:N].set(x)

    fill_s = (N + jnp.arange(E_PAD_S - E, dtype=jnp.int32) % (N_PAD - N))
    src_s = jnp.concatenate([edge_index[0], fill_s]).reshape(NW, NCH_W, CH)
    dst_flat = jnp.concatenate([edge_index[1], fill_s])
    dst_s = dst_flat.reshape(NW, NCH_W, CH)

    zer = jnp.zeros((RPT, D), jnp.float32)

    degp = _deg_hist(dst_flat.reshape(1, E_PAD_T)).reshape(GRID, D, 1)

    bhr = bh.reshape(1, 1)

    hs1 = _mm1(x_pad, W1, degp)

    w_nexts = jnp.stack([W2, W2])
    bs = jnp.stack([b1.reshape(1, D), b2.reshape(1, D)])

    def _step(carry, xs):
        resid, hs = carry
        w_next, b = xs
        a = _sc_scatter(hs, src_s, dst_s, zer)
        h, hs_next = _comb(a, hs, degp, resid, b, w_next)
        return (h, hs_next), None

    (h2, _), _ = lax.scan(_step, (x_pad, hs1), (w_nexts, bs))
    out = _head(h2, Wh.reshape(1, D).T, bhr)
    return out[:N, 0]


# speedup vs baseline: 16.0277x; 1.0172x over previous
import functools

import jax
import jax.numpy as jnp
from jax import lax
from jax.experimental import pallas as pl
from jax.experimental.pallas import tpu as pltpu
from jax.experimental.pallas import tpu_sc as plsc

N = 10000
E = 320000
D = 128

NC = 2
NS = 16
NW = NC * NS
CH = 128
L = 16

NCH_D = -(-E // (NW * CH))
E_PAD_D = NW * NCH_D * CH

GSZ = 16
NG = 5
NCH_W = GSZ * NG
E_PAD_S = NW * NCH_W * CH

N_PAD = 10240
RPT = N_PAD // NS
GRID = N_PAD // 128

_mesh = plsc.VectorSubcoreMesh(core_axis_name="c", subcore_axis_name="s")


@functools.partial(
    pl.kernel,
    out_type=jax.ShapeDtypeStruct((NC, N_PAD, D), jnp.float32),
    mesh=_mesh,
    scratch_types=[
        pltpu.VMEM((GSZ, CH), jnp.int32),
        pltpu.VMEM((GSZ, CH), jnp.int32),
        pltpu.VMEM((GSZ, CH), jnp.int32),
        pltpu.VMEM((GSZ, CH), jnp.int32),
        pltpu.VMEM((CH, D), jnp.float32),
        pltpu.VMEM((CH, D), jnp.float32),
        pltpu.VMEM_SHARED((N_PAD, D), jnp.float32),
        pltpu.SemaphoreType.DMA,
        pltpu.SemaphoreType.DMA,
    ],
)
def _sc_scatter(hs_hbm, src_hbm, dst_hbm, zer_hbm, out_hbm,
                srcA_v, dstA_v, srcB_v, dstB_v, rows0_v, rows1_v,
                acc, gsem, isem):
    c = lax.axis_index("c")
    s = lax.axis_index("s")
    w = c * NS + s
    ibufs = [(srcA_v, dstA_v), (srcB_v, dstB_v)]

    def _iload(g, bufs):
        pltpu.async_copy(src_hbm.at[w, pl.ds(g * GSZ, GSZ)], bufs[0], isem)
        pltpu.async_copy(dst_hbm.at[w, pl.ds(g * GSZ, GSZ)], bufs[1], isem)

    def _iload_wait(g, bufs):
        pltpu.make_async_copy(src_hbm.at[w, pl.ds(g * GSZ, GSZ)], bufs[0], isem).wait()
        pltpu.make_async_copy(dst_hbm.at[w, pl.ds(g * GSZ, GSZ)], bufs[1], isem).wait()

    _iload(0, ibufs[0])

    pltpu.sync_copy(zer_hbm, acc.at[pl.ds(s * RPT, RPT)])
    plsc.subcore_barrier()

    for g in range(NG):
        src_v, dst_v = ibufs[g % 2]

        def _gather(k, buf):
            return pltpu.async_copy(hs_hbm.at[src_v.at[k]], buf, gsem)

        def _gather_wait(k, buf):
            pltpu.make_async_copy(hs_hbm.at[src_v.at[k]], buf, gsem).wait()

        def _scat(k, buf):
            pltpu.sync_copy(buf, acc.at[dst_v.at[k]], add=True)

        _iload_wait(g, ibufs[g % 2])
        if g + 1 < NG:
            _iload(g + 1, ibufs[(g + 1) % 2])

        _gather(0, rows0_v)

        @pl.loop(0, GSZ // 2 - 1)
        def _(i):
            k = 2 * i
            _gather_wait(k, rows0_v)
            _gather(k + 1, rows1_v)
            _scat(k, rows0_v)
            _gather_wait(k + 1, rows1_v)
            _gather(k + 2, rows0_v)
            _scat(k + 1, rows1_v)

        _gather_wait(GSZ - 2, rows0_v)
        _gather(GSZ - 1, rows1_v)
        _scat(GSZ - 2, rows0_v)
        _gather_wait(GSZ - 1, rows1_v)
        _scat(GSZ - 1, rows1_v)

    plsc.subcore_barrier()

    pltpu.sync_copy(acc.at[pl.ds(s * RPT, RPT)],
                    out_hbm.at[c, pl.ds(s * RPT, RPT)])


EB = 4096
E_PAD_T = E_PAD_S
NCH_T = E_PAD_T // EB


def _deg_body(dst_ref, o_ref):
    @pl.when(pl.program_id(0) == 0)
    def _():
        o_ref[...] = jnp.zeros_like(o_ref)

    d = dst_ref[...]
    hi = d >> 7
    lo = d & 127
    uhiT = (lax.broadcasted_iota(jnp.int32, (GRID, EB), 0) == hi).astype(jnp.bfloat16)
    uloT = (lax.broadcasted_iota(jnp.int32, (D, EB), 0) == lo).astype(jnp.bfloat16)
    o_ref[...] += lax.dot_general(uhiT, uloT, (((1,), (1,)), ((), ())),
                                  preferred_element_type=jnp.float32)


def _deg_hist(dst_row):
    return pl.pallas_call(
        _deg_body,
        grid=(NCH_T,),
        in_specs=[pl.BlockSpec((1, EB), lambda i: (0, i))],
        out_specs=pl.BlockSpec((GRID, D), lambda i: (0, 0)),
        out_shape=jax.ShapeDtypeStruct((GRID, D), jnp.float32),
    )(dst_row)


def _dinv_of(dp_ref):
    deg = dp_ref[0] + 1.0
    return lax.rsqrt(deg)


def _mm1_body(x_ref, w_ref, dp_ref, o_ref):
    h = lax.dot_general(x_ref[...], w_ref[...], (((1,), (1,)), ((), ())),
                        preferred_element_type=jnp.float32)
    o_ref[...] = h * _dinv_of(dp_ref)


def _comb_body(a_ref, hs_ref, dp_ref, x_ref, b_ref, w_ref, h_ref, hsn_ref):
    dinv = _dinv_of(dp_ref)
    agg = a_ref[0] + a_ref[1] + hs_ref[...]
    h = jnp.maximum(dinv * agg + b_ref[...] + x_ref[...], 0.0)
    h_ref[...] = h
    hsn_ref[...] = lax.dot_general(h, w_ref[...], (((1,), (1,)), ((), ())),
                                   preferred_element_type=jnp.float32) * dinv


def _head_body(h_ref, wh_ref, bh_ref, o_ref):
    o_ref[...] = lax.dot_general(h_ref[...], wh_ref[...], (((1,), (0,)), ((), ())),
                                 preferred_element_type=jnp.float32) + bh_ref[0, 0]


def _row_spec():
    return pl.BlockSpec((128, D), lambda i: (i, 0))


_A_SPEC = pl.BlockSpec((NC, 128, D), lambda i: (0, i, 0))
_DP_SPEC = pl.BlockSpec((1, 128, 1), lambda i: (i, 0, 0))
_FULL_W = pl.BlockSpec((D, D), lambda i: (0, 0))
_FULL_B = pl.BlockSpec((1, D), lambda i: (0, 0))


def _mm1(x_pad, w, degp):
    return pl.pallas_call(
        _mm1_body,
        grid=(GRID,),
        in_specs=[_row_spec(), _FULL_W, _DP_SPEC],
        out_specs=_row_spec(),
        out_shape=jax.ShapeDtypeStruct((N_PAD, D), jnp.float32),
    )(x_pad, w, degp)


def _comb(agg, hs, degp, resid, b, w_next):
    return pl.pallas_call(
        _comb_body,
        grid=(GRID,),
        in_specs=[_A_SPEC, _row_spec(), _DP_SPEC, _row_spec(), _FULL_B, _FULL_W],
        out_specs=[_row_spec(), _row_spec()],
        out_shape=[jax.ShapeDtypeStruct((N_PAD, D), jnp.float32),
                   jax.ShapeDtypeStruct((N_PAD, D), jnp.float32)],
    )(agg, hs, degp, resid, b, w_next)


def _head(h2, wh, bh):
    return pl.pallas_call(
        _head_body,
        grid=(GRID,),
        in_specs=[_row_spec(),
                  pl.BlockSpec((D, 1), lambda i: (0, 0)),
                  pl.BlockSpec((1, 1), lambda i: (0, 0))],
        out_specs=pl.BlockSpec((128, 1), lambda i: (i, 0)),
        out_shape=jax.ShapeDtypeStruct((N_PAD, 1), jnp.float32),
    )(h2, wh, bh)


def kernel(x, edge_index, W1, b1, W2, b2, Wh, bh):
    x_pad = jnp.zeros((N_PAD, D), jnp.float32).at[:N].set(x)

    fill_s = (N + jnp.arange(E_PAD_S - E, dtype=jnp.int32) % (N_PAD - N))
    src_s = jnp.concatenate([edge_index[0], fill_s]).reshape(NW, NCH_W, CH)
    dst_flat = jnp.concatenate([edge_index[1], fill_s])
    dst_s = dst_flat.reshape(NW, NCH_W, CH)

    zer = jnp.zeros((RPT, D), jnp.float32)

    degp = _deg_hist(dst_flat.reshape(1, E_PAD_T)).reshape(GRID, D, 1)

    bhr = bh.reshape(1, 1)

    hs1 = _mm1(x_pad, W1, degp)

    w_nexts = jnp.stack([W2, W2])
    bs = jnp.stack([b1.reshape(1, D), b2.reshape(1, D)])

    def _step(carry, xs):
        resid, hs = carry
        w_next, b = xs
        a = _sc_scatter(hs, src_s, dst_s, zer)
        h, hs_next = _comb(a, hs, degp, resid, b, w_next)
        return (h, hs_next), None

    (h2, _), _ = lax.scan(_step, (x_pad, hs1), (w_nexts, bs))
    out = _head(h2, Wh.reshape(1, D).T, bhr)
    return out[:N, 0]


# speedup vs baseline: 17.2141x; 1.0740x over previous
import functools

import jax
import jax.numpy as jnp
from jax import lax
from jax.experimental import pallas as pl
from jax.experimental.pallas import tpu as pltpu
from jax.experimental.pallas import tpu_sc as plsc

N = 10000
E = 320000
D = 128

NC = 2
NS = 16
NW = NC * NS
CH = 128
L = 16

NCH_D = -(-E // (NW * CH))
E_PAD_D = NW * NCH_D * CH

GSZ = 16
NG = 5
NCH_W = GSZ * NG
E_PAD_S = NW * NCH_W * CH

N_PAD = 10240
RPT = N_PAD // NS
GRID = N_PAD // 128

_mesh = plsc.VectorSubcoreMesh(core_axis_name="c", subcore_axis_name="s")


@functools.partial(
    pl.kernel,
    out_type=jax.ShapeDtypeStruct((NC, N_PAD, D), jnp.float32),
    mesh=_mesh,
    scratch_types=[
        pltpu.VMEM((GSZ, CH), jnp.int32),
        pltpu.VMEM((GSZ, CH), jnp.int32),
        pltpu.VMEM((GSZ, CH), jnp.int32),
        pltpu.VMEM((GSZ, CH), jnp.int32),
        pltpu.VMEM((CH, D), jnp.float32),
        pltpu.VMEM((CH, D), jnp.float32),
        pltpu.VMEM_SHARED((N_PAD, D), jnp.float32),
        pltpu.SemaphoreType.DMA,
        pltpu.SemaphoreType.DMA,
    ],
)
def _sc_scatter(hs_hbm, src_hbm, dst_hbm, zer_hbm, out_hbm,
                srcA_v, dstA_v, srcB_v, dstB_v, rows0_v, rows1_v,
                acc, gsem, isem):
    c = lax.axis_index("c")
    s = lax.axis_index("s")
    w = c * NS + s
    ibufs = [(srcA_v, dstA_v), (srcB_v, dstB_v)]

    def _iload(g, bufs):
        pltpu.async_copy(src_hbm.at[w, pl.ds(g * GSZ, GSZ)], bufs[0], isem)
        pltpu.async_copy(dst_hbm.at[w, pl.ds(g * GSZ, GSZ)], bufs[1], isem)

    def _iload_wait(g, bufs):
        pltpu.make_async_copy(src_hbm.at[w, pl.ds(g * GSZ, GSZ)], bufs[0], isem).wait()
        pltpu.make_async_copy(dst_hbm.at[w, pl.ds(g * GSZ, GSZ)], bufs[1], isem).wait()

    _iload(0, ibufs[0])

    pltpu.sync_copy(zer_hbm, acc.at[pl.ds(s * RPT, RPT)])
    plsc.subcore_barrier()

    for g in range(NG):
        src_v, dst_v = ibufs[g % 2]

        def _gather(k, buf):
            return pltpu.async_copy(hs_hbm.at[src_v.at[k]], buf, gsem)

        def _gather_wait(k, buf):
            pltpu.make_async_copy(hs_hbm.at[src_v.at[k]], buf, gsem).wait()

        def _scat(k, buf):
            pltpu.sync_copy(buf, acc.at[dst_v.at[k]], add=True)

        _iload_wait(g, ibufs[g % 2])
        if g + 1 < NG:
            _iload(g + 1, ibufs[(g + 1) % 2])

        _gather(0, rows0_v)

        @pl.loop(0, GSZ // 2 - 1)
        def _(i):
            k = 2 * i
            _gather_wait(k, rows0_v)
            _gather(k + 1, rows1_v)
            _scat(k, rows0_v)
            _gather_wait(k + 1, rows1_v)
            _gather(k + 2, rows0_v)
            _scat(k + 1, rows1_v)

        _gather_wait(GSZ - 2, rows0_v)
        _gather(GSZ - 1, rows1_v)
        _scat(GSZ - 2, rows0_v)
        _gather_wait(GSZ - 1, rows1_v)
        _scat(GSZ - 1, rows1_v)

    plsc.subcore_barrier()

    pltpu.sync_copy(acc.at[pl.ds(s * RPT, RPT)],
                    out_hbm.at[c, pl.ds(s * RPT, RPT)])


EB = 4096
E_PAD_T = E_PAD_S
NCH_T = E_PAD_T // EB


def _deg_body(dst_ref, o_ref):
    @pl.when(pl.program_id(0) == 0)
    def _():
        o_ref[...] = jnp.zeros_like(o_ref)

    d = dst_ref[...]
    hi = d >> 7
    lo = d & 127
    uhiT = (lax.broadcasted_iota(jnp.int32, (GRID, EB), 0) == hi).astype(jnp.bfloat16)
    uloT = (lax.broadcasted_iota(jnp.int32, (D, EB), 0) == lo).astype(jnp.bfloat16)
    o_ref[...] += lax.dot_general(uhiT, uloT, (((1,), (1,)), ((), ())),
                                  preferred_element_type=jnp.float32)


def _deg_hist(dst_row):
    return pl.pallas_call(
        _deg_body,
        grid=(NCH_T,),
        in_specs=[pl.BlockSpec((1, EB), lambda i: (0, i))],
        out_specs=pl.BlockSpec((GRID, D), lambda i: (0, 0)),
        out_shape=jax.ShapeDtypeStruct((GRID, D), jnp.float32),
    )(dst_row)


def _dinv_of(dp_ref):
    deg = dp_ref[0] + 1.0
    return lax.rsqrt(deg)


def _mm1_body(x_ref, w_ref, dp_ref, o_ref):
    h = lax.dot_general(x_ref[...], w_ref[...], (((1,), (1,)), ((), ())),
                        preferred_element_type=jnp.float32)
    o_ref[...] = h * _dinv_of(dp_ref)


def _comb_body(a_ref, hs_ref, dp_ref, x_ref, b_ref, w_ref, wh_ref, bh_ref,
               h_ref, hsn_ref, o_ref):
    dinv = _dinv_of(dp_ref)
    agg = a_ref[0] + a_ref[1] + hs_ref[...]
    h = jnp.maximum(dinv * agg + b_ref[...] + x_ref[...], 0.0)
    h_ref[...] = h
    hsn_ref[...] = lax.dot_general(h, w_ref[...], (((1,), (1,)), ((), ())),
                                   preferred_element_type=jnp.float32) * dinv
    o_ref[...] = lax.dot_general(h, wh_ref[...], (((1,), (0,)), ((), ())),
                                 preferred_element_type=jnp.float32) + bh_ref[0, 0]


def _row_spec():
    return pl.BlockSpec((128, D), lambda i: (i, 0))


_A_SPEC = pl.BlockSpec((NC, 128, D), lambda i: (0, i, 0))
_DP_SPEC = pl.BlockSpec((1, 128, 1), lambda i: (i, 0, 0))
_FULL_W = pl.BlockSpec((D, D), lambda i: (0, 0))
_FULL_B = pl.BlockSpec((1, D), lambda i: (0, 0))


def _mm1(x_pad, w, degp):
    return pl.pallas_call(
        _mm1_body,
        grid=(GRID,),
        in_specs=[_row_spec(), _FULL_W, _DP_SPEC],
        out_specs=_row_spec(),
        out_shape=jax.ShapeDtypeStruct((N_PAD, D), jnp.float32),
    )(x_pad, w, degp)


def _comb(agg, hs, degp, resid, b, w_next, wh, bh):
    return pl.pallas_call(
        _comb_body,
        grid=(GRID,),
        in_specs=[_A_SPEC, _row_spec(), _DP_SPEC, _row_spec(), _FULL_B, _FULL_W,
                  pl.BlockSpec((D, 1), lambda i: (0, 0)),
                  pl.BlockSpec((1, 1), lambda i: (0, 0))],
        out_specs=[_row_spec(), _row_spec(),
                   pl.BlockSpec((128, 1), lambda i: (i, 0))],
        out_shape=[jax.ShapeDtypeStruct((N_PAD, D), jnp.float32),
                   jax.ShapeDtypeStruct((N_PAD, D), jnp.float32),
                   jax.ShapeDtypeStruct((N_PAD, 1), jnp.float32)],
    )(agg, hs, degp, resid, b, w_next, wh, bh)


def kernel(x, edge_index, W1, b1, W2, b2, Wh, bh):
    x_pad = jnp.zeros((N_PAD, D), jnp.float32).at[:N].set(x)

    fill_s = (N + jnp.arange(E_PAD_S - E, dtype=jnp.int32) % (N_PAD - N))
    src_s = jnp.concatenate([edge_index[0], fill_s]).reshape(NW, NCH_W, CH)
    dst_flat = jnp.concatenate([edge_index[1], fill_s])
    dst_s = dst_flat.reshape(NW, NCH_W, CH)

    zer = jnp.zeros((RPT, D), jnp.float32)

    degp = _deg_hist(dst_flat.reshape(1, E_PAD_T)).reshape(GRID, D, 1)

    bhr = bh.reshape(1, 1)

    hs1 = _mm1(x_pad, W1, degp)

    w_nexts = jnp.stack([W2, W2])
    bs = jnp.stack([b1.reshape(1, D), b2.reshape(1, D)])

    wh_col = Wh.reshape(1, D).T

    def _step(carry, xs):
        resid, hs = carry
        w_next, b = xs
        a = _sc_scatter(hs, src_s, dst_s, zer)
        h, hs_next, o = _comb(a, hs, degp, resid, b, w_next, wh_col, bhr)
        return (h, hs_next), o

    _, os = lax.scan(_step, (x_pad, hs1), (w_nexts, bs))
    return os[-1, :N, 0]


# speedup vs baseline: 17.2192x; 1.0003x over previous
import functools

import jax
import jax.numpy as jnp
from jax import lax
from jax.experimental import pallas as pl
from jax.experimental.pallas import tpu as pltpu
from jax.experimental.pallas import tpu_sc as plsc

N = 10000
E = 320000
D = 128

NC = 2
NS = 16
NW = NC * NS
CH = 128
L = 16

GSZ = 16
NG = 5
NCH_W = GSZ * NG
E_PAD_S = NW * NCH_W * CH

N_PAD = 10240
RPT = N_PAD // NS
GRID = N_PAD // 128

_mesh = plsc.VectorSubcoreMesh(core_axis_name="c", subcore_axis_name="s")


@functools.partial(
    pl.kernel,
    out_type=jax.ShapeDtypeStruct((NC, N_PAD, D), jnp.float32),
    mesh=_mesh,
    scratch_types=[
        pltpu.VMEM((GSZ, CH), jnp.int32),
        pltpu.VMEM((GSZ, CH), jnp.int32),
        pltpu.VMEM((GSZ, CH), jnp.int32),
        pltpu.VMEM((GSZ, CH), jnp.int32),
        pltpu.VMEM((CH, D), jnp.float32),
        pltpu.VMEM((CH, D), jnp.float32),
        pltpu.VMEM_SHARED((N_PAD, D), jnp.float32),
        pltpu.SemaphoreType.DMA,
        pltpu.SemaphoreType.DMA,
    ],
)
def _sc_scatter(hs_hbm, src_hbm, dst_hbm, zer_hbm, out_hbm,
                srcA_v, dstA_v, srcB_v, dstB_v, rows0_v, rows1_v,
                acc, gsem, isem):
    c = lax.axis_index("c")
    s = lax.axis_index("s")
    w = c * NS + s
    ibufs = [(srcA_v, dstA_v), (srcB_v, dstB_v)]

    def _iload(g, bufs):
        pltpu.async_copy(src_hbm.at[w, pl.ds(g * GSZ, GSZ)], bufs[0], isem)
        pltpu.async_copy(dst_hbm.at[w, pl.ds(g * GSZ, GSZ)], bufs[1], isem)

    def _iload_wait(g, bufs):
        pltpu.make_async_copy(src_hbm.at[w, pl.ds(g * GSZ, GSZ)], bufs[0], isem).wait()
        pltpu.make_async_copy(dst_hbm.at[w, pl.ds(g * GSZ, GSZ)], bufs[1], isem).wait()

    _iload(0, ibufs[0])

    pltpu.sync_copy(zer_hbm, acc.at[pl.ds(s * RPT, RPT)])
    plsc.subcore_barrier()

    for g in range(NG):
        src_v, dst_v = ibufs[g % 2]

        def _gather(k, buf):
            return pltpu.async_copy(hs_hbm.at[src_v.at[k]], buf, gsem)

        def _gather_wait(k, buf):
            pltpu.make_async_copy(hs_hbm.at[src_v.at[k]], buf, gsem).wait()

        def _scat(k, buf):
            pltpu.sync_copy(buf, acc.at[dst_v.at[k]], add=True)

        _iload_wait(g, ibufs[g % 2])
        if g + 1 < NG:
            _iload(g + 1, ibufs[(g + 1) % 2])

        _gather(0, rows0_v)

        @pl.loop(0, GSZ // 2 - 1)
        def _(i):
            k = 2 * i
            _gather_wait(k, rows0_v)
            _gather(k + 1, rows1_v)
            _scat(k, rows0_v)
            _gather_wait(k + 1, rows1_v)
            _gather(k + 2, rows0_v)
            _scat(k + 1, rows1_v)

        _gather_wait(GSZ - 2, rows0_v)
        _gather(GSZ - 1, rows1_v)
        _scat(GSZ - 2, rows0_v)
        _gather_wait(GSZ - 1, rows1_v)
        _scat(GSZ - 1, rows1_v)

    plsc.subcore_barrier()

    pltpu.sync_copy(acc.at[pl.ds(s * RPT, RPT)],
                    out_hbm.at[c, pl.ds(s * RPT, RPT)])


EB = 4096
E_PAD_T = E_PAD_S
NCH_T = E_PAD_T // EB


def _deg_body(dst_ref, o_ref):
    @pl.when(pl.program_id(0) == 0)
    def _():
        o_ref[...] = jnp.zeros_like(o_ref)

    d = dst_ref[...]
    hi = d >> 7
    lo = d & 127
    uhiT = (lax.broadcasted_iota(jnp.int32, (GRID, EB), 0) == hi).astype(jnp.bfloat16)
    uloT = (lax.broadcasted_iota(jnp.int32, (D, EB), 0) == lo).astype(jnp.bfloat16)
    o_ref[...] += lax.dot_general(uhiT, uloT, (((1,), (1,)), ((), ())),
                                  preferred_element_type=jnp.float32)


def _deg_hist(dst_row):
    return pl.pallas_call(
        _deg_body,
        grid=(NCH_T,),
        in_specs=[pl.BlockSpec((1, EB), lambda i: (0, i))],
        out_specs=pl.BlockSpec((GRID, D), lambda i: (0, 0)),
        out_shape=jax.ShapeDtypeStruct((GRID, D), jnp.float32),
    )(dst_row)


def _dinv_of(dp_ref):
    deg = dp_ref[0] + 1.0
    return lax.rsqrt(deg)


def _mm1_body(x_ref, w_ref, dp_ref, o_ref):
    h = lax.dot_general(x_ref[...], w_ref[...], (((1,), (1,)), ((), ())),
                        preferred_element_type=jnp.float32)
    o_ref[...] = h * _dinv_of(dp_ref)


def _comb_body(a_ref, hs_ref, dp_ref, x_ref, b_ref, w_ref, wh_ref, bh_ref,
               h_ref, hsn_ref, o_ref):
    dinv = _dinv_of(dp_ref)
    agg = a_ref[0] + a_ref[1] + hs_ref[...]
    h = jnp.maximum(dinv * agg + b_ref[...] + x_ref[...], 0.0)
    h_ref[...] = h
    hsn_ref[...] = lax.dot_general(h, w_ref[...], (((1,), (1,)), ((), ())),
                                   preferred_element_type=jnp.float32) * dinv
    o_ref[...] = lax.dot_general(h, wh_ref[...], (((1,), (0,)), ((), ())),
                                 preferred_element_type=jnp.float32) + bh_ref[0, 0]


def _row_spec():
    return pl.BlockSpec((128, D), lambda i: (i, 0))


_A_SPEC = pl.BlockSpec((NC, 128, D), lambda i: (0, i, 0))
_DP_SPEC = pl.BlockSpec((1, 128, 1), lambda i: (i, 0, 0))
_FULL_W = pl.BlockSpec((D, D), lambda i: (0, 0))
_FULL_B = pl.BlockSpec((1, D), lambda i: (0, 0))


def _mm1(x_pad, w, degp):
    return pl.pallas_call(
        _mm1_body,
        grid=(GRID,),
        in_specs=[_row_spec(), _FULL_W, _DP_SPEC],
        out_specs=_row_spec(),
        out_shape=jax.ShapeDtypeStruct((N_PAD, D), jnp.float32),
    )(x_pad, w, degp)


def _comb(agg, hs, degp, resid, b, w_next, wh, bh):
    return pl.pallas_call(
        _comb_body,
        grid=(GRID,),
        in_specs=[_A_SPEC, _row_spec(), _DP_SPEC, _row_spec(), _FULL_B, _FULL_W,
                  pl.BlockSpec((D, 1), lambda i: (0, 0)),
                  pl.BlockSpec((1, 1), lambda i: (0, 0))],
        out_specs=[_row_spec(), _row_spec(),
                   pl.BlockSpec((128, 1), lambda i: (i, 0))],
        out_shape=[jax.ShapeDtypeStruct((N_PAD, D), jnp.float32),
                   jax.ShapeDtypeStruct((N_PAD, D), jnp.float32),
                   jax.ShapeDtypeStruct((N_PAD, 1), jnp.float32)],
    )(agg, hs, degp, resid, b, w_next, wh, bh)


def kernel(x, edge_index, W1, b1, W2, b2, Wh, bh):
    x_pad = jnp.zeros((N_PAD, D), jnp.float32).at[:N].set(x)

    fill_s = (N + jnp.arange(E_PAD_S - E, dtype=jnp.int32) % (N_PAD - N))
    src_s = jnp.concatenate([edge_index[0], fill_s]).reshape(NW, NCH_W, CH)
    dst_flat = jnp.concatenate([edge_index[1], fill_s])
    dst_s = dst_flat.reshape(NW, NCH_W, CH)

    zer = jnp.zeros((RPT, D), jnp.float32)

    degp = _deg_hist(dst_flat.reshape(1, E_PAD_T)).reshape(GRID, D, 1)

    bhr = bh.reshape(1, 1)

    hs1 = _mm1(x_pad, W1, degp)

    w_nexts = jnp.stack([W2, W2])
    bs = jnp.stack([b1.reshape(1, D), b2.reshape(1, D)])

    wh_col = Wh.reshape(1, D).T

    def _step(carry, xs):
        resid, hs = carry
        w_next, b = xs
        a = _sc_scatter(hs, src_s, dst_s, zer)
        h, hs_next, o = _comb(a, hs, degp, resid, b, w_next, wh_col, bhr)
        return (h, hs_next), o

    _, os = lax.scan(_step, (x_pad, hs1), (w_nexts, bs))
    return os[-1, :N, 0]
